# baseline (speedup 1.0000x reference)
import numpy as np

import jax
import jax.numpy as jnp
from jax.experimental import pallas as pl
from jax.experimental.pallas import tpu as pltpu
from jax.sharding import Mesh, PartitionSpec as P

_FC = 2048
_NO = 3072
_B = 2048
_BN_EPS = 1e-5
_TB = 512
_CC = 256
_NS = 3


def _leaky(x):
    return jnp.where(x >= 0, x, 0.02 * x)


def _mlp_kernel(z_ref, w1_ref, b1_ref, g1_ref, be1_ref, m1_ref, v1_ref,
                b2_ref, g2_ref, be2_ref, m2_ref, v2_ref, b3_ref,
                w2_hbm, w3_hbm, o_ref,
                w2b, w3b, h2b, stage, sem):
    i = pl.program_id(0)

    s1 = g1_ref[...] * jax.lax.rsqrt(v1_ref[...] + _BN_EPS)
    t1 = be1_ref[...] + (b1_ref[...] - m1_ref[...]) * s1
    s2 = g2_ref[...] * jax.lax.rsqrt(v2_ref[...] + _BN_EPS)
    t2 = be2_ref[...] + (b2_ref[...] - m2_ref[...]) * s2

    zb = z_ref[...].astype(jnp.bfloat16)
    w1 = w1_ref[...].astype(jnp.bfloat16)
    h1 = jnp.dot(zb, w1, preferred_element_type=jnp.float32)
    h1 = _leaky(h1 * s1 + t1).astype(jnp.bfloat16)

    n2 = _FC // _CC
    n3 = _NO // _CC

    @pl.when(i == 0)
    def _stream_and_compute():
        def start(k):
            s = k % _NS
            if k < n2:
                src = w2_hbm.at[:, pl.ds(k * _CC, _CC)]
            else:
                src = w3_hbm.at[:, pl.ds((k - n2) * _CC, _CC)]
            pltpu.make_async_copy(src, stage.at[s], sem.at[s]).start()

        for k in range(_NS):
            start(k)
        for k in range(n2):
            s = k % _NS
            pltpu.make_async_copy(stage.at[s], stage.at[s], sem.at[s]).wait()
            wc = stage[s].astype(jnp.bfloat16)
            w2b[:, pl.ds(k * _CC, _CC)] = wc
            if k + _NS < n2 + n3:
                start(k + _NS)
            hc = jnp.dot(h1, wc, preferred_element_type=jnp.float32)
            hc = hc * s2[:, k * _CC:(k + 1) * _CC] + t2[:, k * _CC:(k + 1) * _CC]
            h2b[:, pl.ds(k * _CC, _CC)] = _leaky(hc).astype(jnp.bfloat16)
        for k in range(n3):
            kk = k + n2
            s = kk % _NS
            pltpu.make_async_copy(stage.at[s], stage.at[s], sem.at[s]).wait()
            wc = stage[s].astype(jnp.bfloat16)
            w3b[:, pl.ds(k * _CC, _CC)] = wc
            if kk + _NS < n2 + n3:
                start(kk + _NS)
            y = jnp.dot(h2b[...], wc, preferred_element_type=jnp.float32)
            o_ref[:, pl.ds(k * _CC, _CC)] = y + b3_ref[:, k * _CC:(k + 1) * _CC]

    @pl.when(i > 0)
    def _steady():
        h2 = jnp.dot(h1, w2b[...], preferred_element_type=jnp.float32)
        h2 = _leaky(h2 * s2 + t2).astype(jnp.bfloat16)
        y = jnp.dot(h2, w3b[...], preferred_element_type=jnp.float32)
        o_ref[...] = y + b3_ref[...]


def _mlp(z, l1_w, l1_b, bn1_g, bn1_b, bn1_m, bn1_v,
         l2_w, l2_b, bn2_g, bn2_b, bn2_m, bn2_v, l3_w, l3_b):
    bl, nz = z.shape

    vec = lambda a: a.reshape(1, -1)
    const = lambda shape: pl.BlockSpec(shape, lambda i: (0, 0))

    return pl.pallas_call(
        _mlp_kernel,
        out_shape=jax.ShapeDtypeStruct((bl, _NO), jnp.float32),
        grid=(bl // _TB,),
        in_specs=[
            pl.BlockSpec((_TB, nz), lambda i: (i, 0)),
            const((nz, _FC)),
            const((1, _FC)), const((1, _FC)), const((1, _FC)),
            const((1, _FC)), const((1, _FC)),
            const((1, _FC)), const((1, _FC)), const((1, _FC)),
            const((1, _FC)), const((1, _FC)),
            const((1, _NO)),
            pl.BlockSpec(memory_space=pl.ANY),
            pl.BlockSpec(memory_space=pl.ANY),
        ],
        out_specs=pl.BlockSpec((_TB, _NO), lambda i: (i, 0)),
        scratch_shapes=[
            pltpu.VMEM((_FC, _FC), jnp.bfloat16),
            pltpu.VMEM((_FC, _NO), jnp.bfloat16),
            pltpu.VMEM((_TB, _FC), jnp.bfloat16),
            pltpu.VMEM((_NS, _FC, _CC), jnp.float32),
            pltpu.SemaphoreType.DMA((_NS,)),
        ],
        compiler_params=pltpu.CompilerParams(
            dimension_semantics=("arbitrary",)),
    )(z, l1_w,
      vec(l1_b), vec(bn1_g), vec(bn1_b), vec(bn1_m), vec(bn1_v),
      vec(l2_b), vec(bn2_g), vec(bn2_b), vec(bn2_m), vec(bn2_v),
      vec(l3_b), l2_w, l3_w)


def kernel(z, l1_w, l1_b, bn1_g, bn1_b, bn1_m, bn1_v,
           l2_w, l2_b, bn2_g, bn2_b, bn2_m, bn2_v, l3_w, l3_b):
    z = z.reshape(_B, -1)
    args = (z, l1_w, l1_b, bn1_g, bn1_b, bn1_m, bn1_v,
            l2_w, l2_b, bn2_g, bn2_b, bn2_m, bn2_v, l3_w, l3_b)

    devs = jax.devices()
    if len(devs) < 2:
        return _mlp(*args)

    mesh = Mesh(np.array(devs[:2]), ("x",))
    specs = (P("x", None),) + (P(),) * 14
    fn = jax.shard_map(_mlp, mesh=mesh, in_specs=specs,
                       out_specs=P("x", None), check_vma=False)
    return fn(*args)


# speedup vs baseline: 8.8235x; 8.8235x over previous
import jax
import jax.numpy as jnp
from jax.experimental import pallas as pl
from jax.experimental.pallas import tpu as pltpu

_FC = 2048
_NO = 3072
_B = 2048
_BN_EPS = 1e-5
_TB = 512
_CC = 256
_NS = 4


def _leaky(x):
    return jnp.where(x >= 0, x, 0.02 * x)


def _mlp_kernel(z_ref, w1_ref, b1_ref, g1_ref, be1_ref, m1_ref, v1_ref,
                b2_ref, g2_ref, be2_ref, m2_ref, v2_ref, b3_ref,
                w2_hbm, w3_hbm, o_ref,
                w2b, w3b, h2b, stage, sem):
    i = pl.program_id(0)
    n2 = _FC // _CC
    n3 = _NO // _CC
    n = n2 + n3

    def start(k):
        s = k % _NS
        if k < n2:
            src = w2_hbm.at[:, pl.ds(k * _CC, _CC)]
        else:
            src = w3_hbm.at[:, pl.ds((k - n2) * _CC, _CC)]
        pltpu.make_async_copy(src, stage.at[s], sem.at[s]).start()

    def wait_cast(k):
        s = k % _NS
        pltpu.make_async_copy(stage.at[s], stage.at[s], sem.at[s]).wait()
        wc = stage[s].astype(jnp.bfloat16)
        if k < n2:
            w2b[:, pl.ds(k * _CC, _CC)] = wc
        else:
            w3b[:, pl.ds((k - n2) * _CC, _CC)] = wc
        if k + _NS < n:
            start(k + _NS)
        return wc

    @pl.when(i == 0)
    def _kick():
        for k in range(_NS):
            start(k)

    s1 = g1_ref[...] * jax.lax.rsqrt(v1_ref[...] + _BN_EPS)
    t1 = be1_ref[...] + (b1_ref[...] - m1_ref[...]) * s1
    s2 = g2_ref[...] * jax.lax.rsqrt(v2_ref[...] + _BN_EPS)
    t2 = be2_ref[...] + (b2_ref[...] - m2_ref[...]) * s2

    zb = z_ref[...].astype(jnp.bfloat16)
    w1 = w1_ref[...].astype(jnp.bfloat16)
    h1 = jnp.dot(zb, w1, preferred_element_type=jnp.float32)
    h1 = _leaky(h1 * s1 + t1).astype(jnp.bfloat16)

    @pl.when(i == 0)
    def _stream_and_compute():
        wc = [None] * n
        wc[0] = wait_cast(0)
        for k in range(n2):
            if k + 1 < n:
                wc[k + 1] = wait_cast(k + 1)
            hc = jnp.dot(h1, wc[k], preferred_element_type=jnp.float32)
            hc = hc * s2[:, k * _CC:(k + 1) * _CC] + t2[:, k * _CC:(k + 1) * _CC]
            h2b[:, pl.ds(k * _CC, _CC)] = _leaky(hc).astype(jnp.bfloat16)
        h2v = h2b[...]
        for k in range(n3):
            kk = k + n2
            if kk + 1 < n:
                wc[kk + 1] = wait_cast(kk + 1)
            y = jnp.dot(h2v, wc[kk], preferred_element_type=jnp.float32)
            o_ref[:, pl.ds(k * _CC, _CC)] = y + b3_ref[:, k * _CC:(k + 1) * _CC]

    @pl.when(i > 0)
    def _steady():
        h2 = jnp.dot(h1, w2b[...], preferred_element_type=jnp.float32)
        h2 = _leaky(h2 * s2 + t2).astype(jnp.bfloat16)
        y = jnp.dot(h2, w3b[...], preferred_element_type=jnp.float32)
        o_ref[...] = y + b3_ref[...]


def kernel(z, l1_w, l1_b, bn1_g, bn1_b, bn1_m, bn1_v,
           l2_w, l2_b, bn2_g, bn2_b, bn2_m, bn2_v, l3_w, l3_b):
    z = z.reshape(_B, -1)
    nz = z.shape[1]

    vec = lambda a: a.reshape(1, -1)
    const = lambda shape: pl.BlockSpec(shape, lambda i: (0, 0))

    return pl.pallas_call(
        _mlp_kernel,
        out_shape=jax.ShapeDtypeStruct((_B, _NO), jnp.float32),
        grid=(_B // _TB,),
        in_specs=[
            pl.BlockSpec((_TB, nz), lambda i: (i, 0)),
            const((nz, _FC)),
            const((1, _FC)), const((1, _FC)), const((1, _FC)),
            const((1, _FC)), const((1, _FC)),
            const((1, _FC)), const((1, _FC)), const((1, _FC)),
            const((1, _FC)), const((1, _FC)),
            const((1, _NO)),
            pl.BlockSpec(memory_space=pl.ANY),
            pl.BlockSpec(memory_space=pl.ANY),
        ],
        out_specs=pl.BlockSpec((_TB, _NO), lambda i: (i, 0)),
        scratch_shapes=[
            pltpu.VMEM((_FC, _FC), jnp.bfloat16),
            pltpu.VMEM((_FC, _NO), jnp.bfloat16),
            pltpu.VMEM((_TB, _FC), jnp.bfloat16),
            pltpu.VMEM((_NS, _FC, _CC), jnp.float32),
            pltpu.SemaphoreType.DMA((_NS,)),
        ],
        compiler_params=pltpu.CompilerParams(
            dimension_semantics=("arbitrary",)),
    )(z, l1_w,
      vec(l1_b), vec(bn1_g), vec(bn1_b), vec(bn1_m), vec(bn1_v),
      vec(l2_b), vec(bn2_g), vec(bn2_b), vec(bn2_m), vec(bn2_v),
      vec(l3_b), l2_w, l3_w)


# speedup vs baseline: 10.2222x; 1.1585x over previous
import jax
import jax.numpy as jnp
from jax.experimental import pallas as pl
from jax.experimental.pallas import tpu as pltpu

_FC = 2048
_NO = 3072
_B = 2048
_BN_EPS = 1e-5
_CC = 256
_N2 = _FC // _CC
_N3 = _NO // _CC


def _leaky(x):
    return jnp.where(x >= 0, x, 0.02 * x)


def _mlp_kernel(z_ref, w1_ref, b1_ref, g1_ref, be1_ref, m1_ref, v1_ref,
                b2_ref, g2_ref, be2_ref, m2_ref, v2_ref, b3_ref,
                w2c_ref, w3c_ref, o_ref, h1b, h2b):
    j = pl.program_id(0)

    @pl.when(j == 0)
    def _layer1():
        s1 = g1_ref[...] * jax.lax.rsqrt(v1_ref[...] + _BN_EPS)
        t1 = be1_ref[...] + (b1_ref[...] - m1_ref[...]) * s1
        zb = z_ref[...].astype(jnp.bfloat16)
        w1 = w1_ref[...].astype(jnp.bfloat16)
        h1 = jnp.dot(zb, w1, preferred_element_type=jnp.float32)
        h1b[...] = _leaky(h1 * s1 + t1).astype(jnp.bfloat16)

    @pl.when(j < _N2)
    def _layer2_chunk():
        s2 = g2_ref[...] * jax.lax.rsqrt(v2_ref[...] + _BN_EPS)
        t2 = be2_ref[...] + (b2_ref[...] - m2_ref[...]) * s2
        wc = w2c_ref[...].astype(jnp.bfloat16)
        hc = jnp.dot(h1b[...], wc, preferred_element_type=jnp.float32)
        col = pl.multiple_of(j * _CC, _CC)
        h2b[:, pl.ds(col, _CC)] = _leaky(hc * s2 + t2).astype(jnp.bfloat16)

    @pl.when(j >= _N2)
    def _layer3_chunk():
        wc = w3c_ref[...].astype(jnp.bfloat16)
        y = jnp.dot(h2b[...], wc, preferred_element_type=jnp.float32)
        o_ref[...] = y + b3_ref[...]


def kernel(z, l1_w, l1_b, bn1_g, bn1_b, bn1_m, bn1_v,
           l2_w, l2_b, bn2_g, bn2_b, bn2_m, bn2_v, l3_w, l3_b):
    z = z.reshape(_B, -1)
    nz = z.shape[1]

    vec = lambda a: a.reshape(1, -1)
    const = lambda shape: pl.BlockSpec(shape, lambda j: (0, 0))
    w2_idx = lambda j: (0, jnp.minimum(j, _N2 - 1))
    w3_idx = lambda j: (0, jnp.maximum(j - _N2, 0))

    return pl.pallas_call(
        _mlp_kernel,
        out_shape=jax.ShapeDtypeStruct((_B, _NO), jnp.float32),
        grid=(_N2 + _N3,),
        in_specs=[
            const((_B, nz)),
            const((nz, _FC)),
            const((1, _FC)), const((1, _FC)), const((1, _FC)),
            const((1, _FC)), const((1, _FC)),
            pl.BlockSpec((1, _CC), w2_idx),
            pl.BlockSpec((1, _CC), w2_idx),
            pl.BlockSpec((1, _CC), w2_idx),
            pl.BlockSpec((1, _CC), w2_idx),
            pl.BlockSpec((1, _CC), w2_idx),
            pl.BlockSpec((1, _CC), w3_idx),
            pl.BlockSpec((_FC, _CC), w2_idx),
            pl.BlockSpec((_FC, _CC), w3_idx),
        ],
        out_specs=pl.BlockSpec((_B, _CC), w3_idx),
        scratch_shapes=[
            pltpu.VMEM((_B, _FC), jnp.bfloat16),
            pltpu.VMEM((_B, _FC), jnp.bfloat16),
        ],
        compiler_params=pltpu.CompilerParams(
            dimension_semantics=("arbitrary",)),
    )(z, l1_w,
      vec(l1_b), vec(bn1_g), vec(bn1_b), vec(bn1_m), vec(bn1_v),
      vec(l2_b), vec(bn2_g), vec(bn2_b), vec(bn2_m), vec(bn2_v),
      vec(l3_b), l2_w, l3_w)


# speedup vs baseline: 10.4896x; 1.0262x over previous
import jax
import jax.numpy as jnp
from jax.experimental import pallas as pl
from jax.experimental.pallas import tpu as pltpu

_FC = 2048
_NO = 3072
_B = 2048
_BN_EPS = 1e-5
_CC = 512
_N2 = _FC // _CC
_N3 = _NO // _CC


def _leaky(x):
    return jnp.where(x >= 0, x, 0.02 * x)


def _mlp_kernel(z_ref, w1_ref, b1_ref, g1_ref, be1_ref, m1_ref, v1_ref,
                b2_ref, g2_ref, be2_ref, m2_ref, v2_ref, b3_ref,
                w2c_ref, w3c_ref, o_ref, h1b, h2b):
    j = pl.program_id(0)

    @pl.when(j == 0)
    def _layer1():
        s1 = g1_ref[...] * jax.lax.rsqrt(v1_ref[...] + _BN_EPS)
        t1 = be1_ref[...] + (b1_ref[...] - m1_ref[...]) * s1
        zb = z_ref[...].astype(jnp.bfloat16)
        w1 = w1_ref[...].astype(jnp.bfloat16)
        h1 = jnp.dot(zb, w1, preferred_element_type=jnp.float32)
        h1b[...] = _leaky(h1 * s1 + t1).astype(jnp.bfloat16)

    @pl.when(j < _N2)
    def _layer2_chunk():
        s2 = g2_ref[...] * jax.lax.rsqrt(v2_ref[...] + _BN_EPS)
        t2 = be2_ref[...] + (b2_ref[...] - m2_ref[...]) * s2
        wc = w2c_ref[...].astype(jnp.bfloat16)
        hc = jnp.dot(h1b[...], wc, preferred_element_type=jnp.float32)
        col = pl.multiple_of(j * _CC, _CC)
        h2b[:, pl.ds(col, _CC)] = _leaky(hc * s2 + t2).astype(jnp.bfloat16)

    @pl.when(j >= _N2)
    def _layer3_chunk():
        wc = w3c_ref[...].astype(jnp.bfloat16)
        y = jnp.dot(h2b[...], wc, preferred_element_type=jnp.float32)
        o_ref[...] = y + b3_ref[...]


def kernel(z, l1_w, l1_b, bn1_g, bn1_b, bn1_m, bn1_v,
           l2_w, l2_b, bn2_g, bn2_b, bn2_m, bn2_v, l3_w, l3_b):
    z = z.reshape(_B, -1)
    nz = z.shape[1]

    vec = lambda a: a.reshape(1, -1)
    const = lambda shape: pl.BlockSpec(shape, lambda j: (0, 0))
    w2_idx = lambda j: (0, jnp.minimum(j, _N2 - 1))
    w3_idx = lambda j: (0, jnp.maximum(j - _N2, 0))

    return pl.pallas_call(
        _mlp_kernel,
        out_shape=jax.ShapeDtypeStruct((_B, _NO), jnp.float32),
        grid=(_N2 + _N3,),
        in_specs=[
            const((_B, nz)),
            const((nz, _FC)),
            const((1, _FC)), const((1, _FC)), const((1, _FC)),
            const((1, _FC)), const((1, _FC)),
            pl.BlockSpec((1, _CC), w2_idx),
            pl.BlockSpec((1, _CC), w2_idx),
            pl.BlockSpec((1, _CC), w2_idx),
            pl.BlockSpec((1, _CC), w2_idx),
            pl.BlockSpec((1, _CC), w2_idx),
            pl.BlockSpec((1, _CC), w3_idx),
            pl.BlockSpec((_FC, _CC), w2_idx),
            pl.BlockSpec((_FC, _CC), w3_idx),
        ],
        out_specs=pl.BlockSpec((_B, _CC), w3_idx),
        scratch_shapes=[
            pltpu.VMEM((_B, _FC), jnp.bfloat16),
            pltpu.VMEM((_B, _FC), jnp.bfloat16),
        ],
        compiler_params=pltpu.CompilerParams(
            dimension_semantics=("arbitrary",)),
    )(z, l1_w,
      vec(l1_b), vec(bn1_g), vec(bn1_b), vec(bn1_m), vec(bn1_v),
      vec(l2_b), vec(bn2_g), vec(bn2_b), vec(bn2_m), vec(bn2_v),
      vec(l3_b), l2_w, l3_w)


# speedup vs baseline: 10.8226x; 1.0317x over previous
import jax
import jax.numpy as jnp
from jax.experimental import pallas as pl
from jax.experimental.pallas import tpu as pltpu

_FC = 2048
_NO = 3072
_B = 2048
_BN_EPS = 1e-5
_CC = 512
_N2 = _FC // _CC
_N3 = _NO // _CC
_NSL = 3


def _leaky(x):
    return jnp.where(x >= 0, x, 0.02 * x)


def _mlp_kernel(z_ref, w1_ref, b1_ref, g1_ref, be1_ref, m1_ref, v1_ref,
                b2_ref, g2_ref, be2_ref, m2_ref, v2_ref, b3_ref,
                w2c_ref, w3_hbm, o_ref, h1b, h2b, w3st, sem):
    j = pl.program_id(0)

    def w3_start(c):
        pltpu.make_async_copy(
            w3_hbm.at[:, pl.ds(c * _CC, _CC)],
            w3st.at[c % _NSL], sem.at[c % _NSL]).start()

    @pl.when(j == 0)
    def _layer1_and_prefetch():
        for c in range(_NSL):
            w3_start(c)
        s1 = g1_ref[...] * jax.lax.rsqrt(v1_ref[...] + _BN_EPS)
        t1 = be1_ref[...] + (b1_ref[...] - m1_ref[...]) * s1
        zb = z_ref[...].astype(jnp.bfloat16)
        w1 = w1_ref[...].astype(jnp.bfloat16)
        h1 = jnp.dot(zb, w1, preferred_element_type=jnp.float32)
        h1b[...] = _leaky(h1 * s1 + t1).astype(jnp.bfloat16)

    @pl.when(j < _N2)
    def _layer2_chunk():
        s2 = g2_ref[...] * jax.lax.rsqrt(v2_ref[...] + _BN_EPS)
        t2 = be2_ref[...] + (b2_ref[...] - m2_ref[...]) * s2
        wc = w2c_ref[...].astype(jnp.bfloat16)
        hc = jnp.dot(h1b[...], wc, preferred_element_type=jnp.float32)
        col = pl.multiple_of(j * _CC, _CC)
        h2b[:, pl.ds(col, _CC)] = _leaky(hc * s2 + t2).astype(jnp.bfloat16)

    for c in range(_N3):
        @pl.when(j == _N2 + c)
        def _layer3_chunk(c=c):
            s = c % _NSL
            pltpu.make_async_copy(w3st.at[s], w3st.at[s], sem.at[s]).wait()
            wc = w3st[s].astype(jnp.bfloat16)
            if c + _NSL < _N3:
                w3_start(c + _NSL)
            y = jnp.dot(h2b[...], wc, preferred_element_type=jnp.float32)
            o_ref[...] = y + b3_ref[...]


def kernel(z, l1_w, l1_b, bn1_g, bn1_b, bn1_m, bn1_v,
           l2_w, l2_b, bn2_g, bn2_b, bn2_m, bn2_v, l3_w, l3_b):
    z = z.reshape(_B, -1)
    nz = z.shape[1]

    vec = lambda a: a.reshape(1, -1)
    const = lambda shape: pl.BlockSpec(shape, lambda j: (0, 0))
    w2_idx = lambda j: (0, jnp.minimum(j, _N2 - 1))
    w3_idx = lambda j: (0, jnp.maximum(j - _N2, 0))

    return pl.pallas_call(
        _mlp_kernel,
        out_shape=jax.ShapeDtypeStruct((_B, _NO), jnp.float32),
        grid=(_N2 + _N3,),
        in_specs=[
            const((_B, nz)),
            const((nz, _FC)),
            const((1, _FC)), const((1, _FC)), const((1, _FC)),
            const((1, _FC)), const((1, _FC)),
            pl.BlockSpec((1, _CC), w2_idx),
            pl.BlockSpec((1, _CC), w2_idx),
            pl.BlockSpec((1, _CC), w2_idx),
            pl.BlockSpec((1, _CC), w2_idx),
            pl.BlockSpec((1, _CC), w2_idx),
            pl.BlockSpec((1, _CC), w3_idx),
            pl.BlockSpec((_FC, _CC), w2_idx),
            pl.BlockSpec(memory_space=pl.ANY),
        ],
        out_specs=pl.BlockSpec((_B, _CC), w3_idx),
        scratch_shapes=[
            pltpu.VMEM((_B, _FC), jnp.bfloat16),
            pltpu.VMEM((_B, _FC), jnp.bfloat16),
            pltpu.VMEM((_NSL, _FC, _CC), jnp.float32),
            pltpu.SemaphoreType.DMA((_NSL,)),
        ],
        compiler_params=pltpu.CompilerParams(
            dimension_semantics=("arbitrary",)),
    )(z, l1_w,
      vec(l1_b), vec(bn1_g), vec(bn1_b), vec(bn1_m), vec(bn1_v),
      vec(l2_b), vec(bn2_g), vec(bn2_b), vec(bn2_m), vec(bn2_v),
      vec(l3_b), l2_w, l3_w)


# speedup vs baseline: 10.8513x; 1.0026x over previous
import jax
import jax.numpy as jnp
from jax.experimental import pallas as pl
from jax.experimental.pallas import tpu as pltpu

_FC = 2048
_NO = 3072
_B = 2048
_BN_EPS = 1e-5
_CC = 512
_N2 = _FC // _CC
_N3 = _NO // _CC
_NSL = 3


def _leaky(x):
    return jnp.where(x >= 0, x, 0.02 * x)


def _mlp_kernel(z_ref, w1_ref, b1_ref, g1_ref, be1_ref, m1_ref, v1_ref,
                b2_ref, g2_ref, be2_ref, m2_ref, v2_ref, b3_ref,
                w2c_ref, w3_hbm, o_ref, h1b, h2b, w3st, sem):
    j = pl.program_id(0)

    def w3_start(c):
        pltpu.make_async_copy(
            w3_hbm.at[:, pl.ds(c * _CC, _CC)],
            w3st.at[c % _NSL], sem.at[c % _NSL]).start()

    for c in range(_NSL):
        @pl.when(j == c)
        def _prefetch(c=c):
            w3_start(c)

    @pl.when(j == 0)
    def _layer1():
        s1 = g1_ref[...] * jax.lax.rsqrt(v1_ref[...] + _BN_EPS)
        t1 = be1_ref[...] + (b1_ref[...] - m1_ref[...]) * s1
        zb = z_ref[...].astype(jnp.bfloat16)
        w1 = w1_ref[...].astype(jnp.bfloat16)
        h1 = jnp.dot(zb, w1, preferred_element_type=jnp.float32)
        h1b[...] = _leaky(h1 * s1 + t1).astype(jnp.bfloat16)

    @pl.when(j < _N2)
    def _layer2_chunk():
        s2 = g2_ref[...] * jax.lax.rsqrt(v2_ref[...] + _BN_EPS)
        t2 = be2_ref[...] + (b2_ref[...] - m2_ref[...]) * s2
        wc = w2c_ref[...].astype(jnp.bfloat16)
        hc = jnp.dot(h1b[...], wc, preferred_element_type=jnp.float32)
        col = pl.multiple_of(j * _CC, _CC)
        h2b[:, pl.ds(col, _CC)] = _leaky(hc * s2 + t2).astype(jnp.bfloat16)

    for c in range(_N3):
        @pl.when(j == _N2 + c)
        def _layer3_chunk(c=c):
            s = c % _NSL
            pltpu.make_async_copy(w3st.at[s], w3st.at[s], sem.at[s]).wait()
            wc = w3st[s].astype(jnp.bfloat16)
            if c + _NSL < _N3:
                w3_start(c + _NSL)
            y = jnp.dot(h2b[...], wc, preferred_element_type=jnp.float32)
            o_ref[...] = y + b3_ref[...]


def kernel(z, l1_w, l1_b, bn1_g, bn1_b, bn1_m, bn1_v,
           l2_w, l2_b, bn2_g, bn2_b, bn2_m, bn2_v, l3_w, l3_b):
    z = z.reshape(_B, -1)
    nz = z.shape[1]

    vec = lambda a: a.reshape(1, -1)
    const = lambda shape: pl.BlockSpec(shape, lambda j: (0, 0))
    w2_idx = lambda j: (0, jnp.minimum(j, _N2 - 1))
    w3_idx = lambda j: (0, jnp.maximum(j - _N2, 0))

    return pl.pallas_call(
        _mlp_kernel,
        out_shape=jax.ShapeDtypeStruct((_B, _NO), jnp.float32),
        grid=(_N2 + _N3,),
        in_specs=[
            const((_B, nz)),
            const((nz, _FC)),
            const((1, _FC)), const((1, _FC)), const((1, _FC)),
            const((1, _FC)), const((1, _FC)),
            pl.BlockSpec((1, _CC), w2_idx),
            pl.BlockSpec((1, _CC), w2_idx),
            pl.BlockSpec((1, _CC), w2_idx),
            pl.BlockSpec((1, _CC), w2_idx),
            pl.BlockSpec((1, _CC), w2_idx),
            pl.BlockSpec((1, _CC), w3_idx),
            pl.BlockSpec((_FC, _CC), w2_idx),
            pl.BlockSpec(memory_space=pl.ANY),
        ],
        out_specs=pl.BlockSpec((_B, _CC), w3_idx),
        scratch_shapes=[
            pltpu.VMEM((_B, _FC), jnp.bfloat16),
            pltpu.VMEM((_B, _FC), jnp.bfloat16),
            pltpu.VMEM((_NSL, _FC, _CC), jnp.float32),
            pltpu.SemaphoreType.DMA((_NSL,)),
        ],
        compiler_params=pltpu.CompilerParams(
            dimension_semantics=("arbitrary",)),
    )(z, l1_w,
      vec(l1_b), vec(bn1_g), vec(bn1_b), vec(bn1_m), vec(bn1_v),
      vec(l2_b), vec(bn2_g), vec(bn2_b), vec(bn2_m), vec(bn2_v),
      vec(l3_b), l2_w, l3_w)
